# Initial kernel scaffold; baseline (speedup 1.0000x reference)
import functools

import jax
import jax.numpy as jnp
from jax import lax
from jax.experimental import pallas as pl
from jax.experimental.pallas import tpu as pltpu
from jax.experimental.pallas import tpu_sc as plsc

N = 10000
E = 320000
D = 128
G = 64

NC = 2
NS = 16
NW = NC * NS
EPW = E // NW
CH = 80
NCHUNK = EPW // CH
RPT = N // NS

_sc_mesh = plsc.VectorSubcoreMesh(core_axis_name="c", subcore_axis_name="s")


@functools.partial(
    pl.kernel,
    out_type=jax.ShapeDtypeStruct((NC * N, D), jnp.float32),
    mesh=_sc_mesh,
    scratch_types=[
        pltpu.VMEM((NCHUNK, CH), jnp.int32),
        pltpu.VMEM((NCHUNK, CH), jnp.int32),
        pltpu.VMEM((CH, D), jnp.float32),
        pltpu.VMEM_SHARED((N, D), jnp.float32),
        pltpu.SemaphoreType.DMA,
    ],
)
def _sc_segment_sum(h_hbm, src_hbm, dst_hbm, zeros_hbm, out_hbm,
                    srcbuf, dstbuf, rows, acc, sem):
    c = lax.axis_index("c")
    s = lax.axis_index("s")
    w = s * NC + c
    pltpu.sync_copy(src_hbm.at[w], srcbuf)
    pltpu.sync_copy(dst_hbm.at[w], dstbuf)
    pltpu.sync_copy(zeros_hbm, acc.at[pl.ds(s * RPT, RPT)])
    plsc.subcore_barrier()

    def body(j, carry):
        pltpu.async_copy(h_hbm.at[srcbuf.at[j]], rows, sem).wait()
        pltpu.sync_copy(rows, acc.at[dstbuf.at[j]], add=True)
        return carry

    lax.fori_loop(0, NCHUNK, body, 0)
    plsc.subcore_barrier()
    pltpu.sync_copy(acc.at[pl.ds(s * RPT, RPT)],
                    out_hbm.at[pl.ds(c * N + s * RPT, RPT)])


BN = 2000
NBLK = N // BN


def _dense_body(relu, a_ref, h_ref, wl_ref, wr_ref, b_ref, o_ref):
    agg = a_ref[0] + a_ref[1]
    y = (jnp.dot(agg, wl_ref[...], preferred_element_type=jnp.float32)
         + jnp.dot(h_ref[...], wr_ref[...], preferred_element_type=jnp.float32)
         + b_ref[...])
    o_ref[...] = jnp.maximum(y, 0.0) if relu else y


def _dense_layer(a2, h, wl, wr, b, relu):
    body = functools.partial(_dense_body, relu)
    return pl.pallas_call(
        body,
        grid=(NBLK,),
        in_specs=[
            pl.BlockSpec((2, BN, D), lambda i: (0, i, 0)),
            pl.BlockSpec((BN, D), lambda i: (i, 0)),
            pl.BlockSpec((D, D), lambda i: (0, 0)),
            pl.BlockSpec((D, D), lambda i: (0, 0)),
            pl.BlockSpec((1, D), lambda i: (0, 0)),
        ],
        out_specs=pl.BlockSpec((BN, D), lambda i: (i, 0)),
        out_shape=jax.ShapeDtypeStruct((N, D), jnp.float32),
    )(a2, h, wl, wr, b)


def _final_body(a_ref, h_ref, wl_ref, wr_ref, b_ref, batch_ref,
                wlin_ref, blin_ref, o_ref):
    i = pl.program_id(0)

    @pl.when(i == 0)
    def _():
        o_ref[...] = jnp.zeros_like(o_ref)

    agg = a_ref[0] + a_ref[1]
    h3 = (jnp.dot(agg, wl_ref[...], preferred_element_type=jnp.float32)
          + jnp.dot(h_ref[...], wr_ref[...], preferred_element_type=jnp.float32)
          + b_ref[...])
    seg = batch_ref[0]
    onehot = (lax.broadcasted_iota(jnp.int32, (G, BN), 0) == seg).astype(jnp.float32)
    o_ref[...] += jnp.dot(onehot, h3, preferred_element_type=jnp.float32)

    @pl.when(i == NBLK - 1)
    def _():
        o_ref[...] = (jnp.dot(o_ref[...], wlin_ref[...],
                              preferred_element_type=jnp.float32)
                      + blin_ref[...])


def _final_layer(a2, h, wl, wr, b, batch3, wlin_pad, blin_pad):
    return pl.pallas_call(
        _final_body,
        grid=(NBLK,),
        in_specs=[
            pl.BlockSpec((2, BN, D), lambda i: (0, i, 0)),
            pl.BlockSpec((BN, D), lambda i: (i, 0)),
            pl.BlockSpec((D, D), lambda i: (0, 0)),
            pl.BlockSpec((D, D), lambda i: (0, 0)),
            pl.BlockSpec((1, D), lambda i: (0, 0)),
            pl.BlockSpec((1, 1, BN), lambda i: (i, 0, 0)),
            pl.BlockSpec((D, D), lambda i: (0, 0)),
            pl.BlockSpec((1, D), lambda i: (0, 0)),
        ],
        out_specs=pl.BlockSpec((G, D), lambda i: (0, 0)),
        out_shape=jax.ShapeDtypeStruct((G, D), jnp.float32),
    )(a2, h, wl, wr, b, batch3, wlin_pad, blin_pad)


def kernel(x, edge_index, batch, Wl1, Wr1, b1, Wl2, Wr2, b2, Wl3, Wr3, b3,
           Wlin, blin):
    src3 = edge_index[0].reshape(NW, NCHUNK, CH)
    dst3 = edge_index[1].reshape(NW, NCHUNK, CH)
    zeros = jnp.zeros((RPT, D), jnp.float32)
    batch3 = batch.reshape(NBLK, 1, BN)
    b1r = b1.reshape(1, D)
    b2r = b2.reshape(1, D)
    b3r = b3.reshape(1, D)
    wlin_pad = jnp.zeros((D, D), jnp.float32).at[:, : Wlin.shape[1]].set(Wlin)
    blin_pad = jnp.zeros((1, D), jnp.float32).at[0, : blin.shape[0]].set(blin)

    h = x
    a = _sc_segment_sum(h, src3, dst3, zeros).reshape(2, N, D)
    h = _dense_layer(a, h, Wl1, Wr1, b1r, relu=True)
    a = _sc_segment_sum(h, src3, dst3, zeros).reshape(2, N, D)
    h = _dense_layer(a, h, Wl2, Wr2, b2r, relu=True)
    a = _sc_segment_sum(h, src3, dst3, zeros).reshape(2, N, D)
    out = _final_layer(a, h, Wl3, Wr3, b3r, batch3, wlin_pad, blin_pad)
    return out[:, : Wlin.shape[1]]


# speedup vs baseline: 7.3534x; 7.3534x over previous
import functools

import jax
import jax.numpy as jnp
from jax import lax
from jax.experimental import pallas as pl
from jax.experimental.pallas import tpu as pltpu
from jax.experimental.pallas import tpu_sc as plsc

N = 10000
NP = 10240
E = 320000
D = 128
G = 64

NC = 2
NS = 16
NW = NC * NS
EPW = E // NW
CH = 80
NCHUNK = EPW // CH
RPT = NP // NS

_sc_mesh = plsc.VectorSubcoreMesh(core_axis_name="c", subcore_axis_name="s")


@functools.partial(
    pl.kernel,
    out_type=jax.ShapeDtypeStruct((NC * NP, D), jnp.float32),
    mesh=_sc_mesh,
    scratch_types=[
        pltpu.VMEM((NCHUNK, CH), jnp.int32),
        pltpu.VMEM((NCHUNK, CH), jnp.int32),
        pltpu.VMEM((CH, D), jnp.float32),
        pltpu.VMEM_SHARED((NP, D), jnp.float32),
        pltpu.SemaphoreType.DMA,
    ],
)
def _sc_segment_sum(h_hbm, src_hbm, dst_hbm, zeros_hbm, out_hbm,
                    srcbuf, dstbuf, rows, acc, sem):
    c = lax.axis_index("c")
    s = lax.axis_index("s")
    w = s * NC + c
    pltpu.sync_copy(src_hbm.at[w], srcbuf)
    pltpu.sync_copy(dst_hbm.at[w], dstbuf)
    pltpu.sync_copy(zeros_hbm, acc.at[pl.ds(s * RPT, RPT)])
    plsc.subcore_barrier()

    def body(j, carry):
        pltpu.async_copy(h_hbm.at[srcbuf.at[j]], rows, sem).wait()
        pltpu.sync_copy(rows, acc.at[dstbuf.at[j]], add=True)
        return carry

    lax.fori_loop(0, NCHUNK, body, 0)
    plsc.subcore_barrier()
    pltpu.sync_copy(acc.at[pl.ds(s * RPT, RPT)],
                    out_hbm.at[pl.ds(c * NP + s * RPT, RPT)])


BN = 2048
NBLK = NP // BN


def _dense_body(relu, a_ref, h_ref, wl_ref, wr_ref, b_ref, o_ref):
    agg = a_ref[0] + a_ref[1]
    y = (jnp.dot(agg, wl_ref[...], preferred_element_type=jnp.float32)
         + jnp.dot(h_ref[...], wr_ref[...], preferred_element_type=jnp.float32)
         + b_ref[...])
    o_ref[...] = jnp.maximum(y, 0.0) if relu else y


def _dense_layer(a2, h, wl, wr, b, relu):
    body = functools.partial(_dense_body, relu)
    return pl.pallas_call(
        body,
        grid=(NBLK,),
        in_specs=[
            pl.BlockSpec((2, BN, D), lambda i: (0, i, 0)),
            pl.BlockSpec((BN, D), lambda i: (i, 0)),
            pl.BlockSpec((D, D), lambda i: (0, 0)),
            pl.BlockSpec((D, D), lambda i: (0, 0)),
            pl.BlockSpec((1, D), lambda i: (0, 0)),
        ],
        out_specs=pl.BlockSpec((BN, D), lambda i: (i, 0)),
        out_shape=jax.ShapeDtypeStruct((NP, D), jnp.float32),
    )(a2, h, wl, wr, b)


def _final_body(a_ref, h_ref, wl_ref, wr_ref, b_ref, batch_ref,
                wlin_ref, blin_ref, o_ref):
    i = pl.program_id(0)

    @pl.when(i == 0)
    def _():
        o_ref[...] = jnp.zeros_like(o_ref)

    agg = a_ref[0] + a_ref[1]
    h3 = (jnp.dot(agg, wl_ref[...], preferred_element_type=jnp.float32)
          + jnp.dot(h_ref[...], wr_ref[...], preferred_element_type=jnp.float32)
          + b_ref[...])
    seg = batch_ref[0]
    onehot = (lax.broadcasted_iota(jnp.int32, (G, BN), 0) == seg).astype(jnp.float32)
    o_ref[...] += jnp.dot(onehot, h3, preferred_element_type=jnp.float32)

    @pl.when(i == NBLK - 1)
    def _():
        o_ref[...] = (jnp.dot(o_ref[...], wlin_ref[...],
                              preferred_element_type=jnp.float32)
                      + blin_ref[...])


def _final_layer(a2, h, wl, wr, b, batch3, wlin_pad, blin_pad):
    return pl.pallas_call(
        _final_body,
        grid=(NBLK,),
        in_specs=[
            pl.BlockSpec((2, BN, D), lambda i: (0, i, 0)),
            pl.BlockSpec((BN, D), lambda i: (i, 0)),
            pl.BlockSpec((D, D), lambda i: (0, 0)),
            pl.BlockSpec((D, D), lambda i: (0, 0)),
            pl.BlockSpec((1, D), lambda i: (0, 0)),
            pl.BlockSpec((1, 1, BN), lambda i: (i, 0, 0)),
            pl.BlockSpec((D, D), lambda i: (0, 0)),
            pl.BlockSpec((1, D), lambda i: (0, 0)),
        ],
        out_specs=pl.BlockSpec((G, D), lambda i: (0, 0)),
        out_shape=jax.ShapeDtypeStruct((G, D), jnp.float32),
    )(a2, h, wl, wr, b, batch3, wlin_pad, blin_pad)


def kernel(x, edge_index, batch, Wl1, Wr1, b1, Wl2, Wr2, b2, Wl3, Wr3, b3,
           Wlin, blin):
    src3 = edge_index[0].reshape(NW, NCHUNK, CH)
    dst3 = edge_index[1].reshape(NW, NCHUNK, CH)
    zeros = jnp.zeros((RPT, D), jnp.float32)
    batch3 = jnp.full((NBLK * BN,), G, jnp.int32).at[:N].set(batch).reshape(NBLK, 1, BN)
    b1r = b1.reshape(1, D)
    b2r = b2.reshape(1, D)
    b3r = b3.reshape(1, D)
    wlin_pad = jnp.zeros((D, D), jnp.float32).at[:, : Wlin.shape[1]].set(Wlin)
    blin_pad = jnp.zeros((1, D), jnp.float32).at[0, : blin.shape[0]].set(blin)

    h = jnp.zeros((NP, D), jnp.float32).at[:N].set(x)
    a = _sc_segment_sum(h, src3, dst3, zeros).reshape(2, NP, D)
    h = _dense_layer(a, h, Wl1, Wr1, b1r, relu=True)
    a = _sc_segment_sum(h, src3, dst3, zeros).reshape(2, NP, D)
    h = _dense_layer(a, h, Wl2, Wr2, b2r, relu=True)
    a = _sc_segment_sum(h, src3, dst3, zeros).reshape(2, NP, D)
    out = _final_layer(a, h, Wl3, Wr3, b3r, batch3, wlin_pad, blin_pad)
    return out[:, : Wlin.shape[1]]
